# baseline (speedup 1.0000x reference)
import functools

import jax
import jax.numpy as jnp
from jax import lax
from jax.experimental import pallas as pl
from jax.experimental.pallas import tpu as pltpu
from jax.experimental.pallas import tpu_sc as plsc

_L = 16
_NBUF = 4
_AHEAD = 2


@functools.lru_cache(maxsize=None)
def _make_scan(n_rows, n_cols, nc=2, ns=16, chunk_rows=4):
    nw = nc * ns
    rows_per_w = n_rows // nw
    n_chunks = rows_per_w // chunk_rows
    assert n_chunks % _NBUF == 0 and n_chunks >= 2 * _NBUF
    vregs_per_row = n_cols // _L
    mesh = plsc.VectorSubcoreMesh(core_axis_name="c", subcore_axis_name="s")

    @functools.partial(
        pl.kernel,
        out_type=jax.ShapeDtypeStruct((n_rows, n_cols), jnp.float32),
        mesh=mesh,
        scratch_types=(
            [pltpu.VMEM((chunk_rows, n_cols), jnp.float32)] * _NBUF
            + [pltpu.SemaphoreType.DMA] * (2 * _NBUF)
        ),
        compiler_params=pltpu.CompilerParams(needs_layout_passes=False),
    )
    def scan_k(x_hbm, out_hbm, *scratch):
        bufs = scratch[:_NBUF]
        lsems = scratch[_NBUF:2 * _NBUF]
        ssems = scratch[2 * _NBUF:]
        wid = lax.axis_index("s") * nc + lax.axis_index("c")
        row_base = wid * rows_per_w
        max_r0 = row_base + (n_chunks - 1) * chunk_rows
        idx15 = jnp.full((_L,), _L - 1, jnp.int32)

        def compute(buf):
            def pair_body(p, _):
                r0 = 2 * p
                r1 = r0 + 1

                def vec_body(j, carries):
                    ca, cb = carries
                    o = j * _L
                    sa = plsc.cumsum(buf[r0, pl.ds(o, _L)])
                    sb = plsc.cumsum(buf[r1, pl.ds(o, _L)])
                    buf[r0, pl.ds(o, _L)] = sa + ca
                    buf[r1, pl.ds(o, _L)] = sb + cb
                    return (ca + sa.at[idx15].get(mode="promise_in_bounds"),
                            cb + sb.at[idx15].get(mode="promise_in_bounds"))

                z = jnp.zeros((_L,), jnp.float32)
                lax.fori_loop(0, vregs_per_row, vec_body, (z, z), unroll=4)
                return 0

            lax.fori_loop(0, chunk_rows // 2, pair_body, 0)

        def wait_load(b):
            pltpu.make_async_copy(
                x_hbm.at[pl.ds(row_base, chunk_rows)], bufs[b],
                lsems[b]).wait()

        def wait_store(b):
            pltpu.make_async_copy(
                bufs[b], out_hbm.at[pl.ds(row_base, chunk_rows)],
                ssems[b]).wait()

        for b in range(_AHEAD):
            pltpu.async_copy(
                x_hbm.at[pl.ds(row_base + b * chunk_rows, chunk_rows)],
                bufs[b], lsems[b])

        def group_body(k, _):
            for b in range(_NBUF):
                r0 = row_base + (k * _NBUF + b) * chunk_rows
                wait_load(b)
                compute(bufs[b])
                pltpu.async_copy(bufs[b],
                                 out_hbm.at[pl.ds(r0, chunk_rows)], ssems[b])
                b2 = (b + _AHEAD) % _NBUF
                if b + _AHEAD >= _NBUF:
                    wait_store(b2)
                else:
                    @pl.when(k > 0)
                    def _():
                        wait_store(b2)
                nxt = jnp.minimum(r0 + _AHEAD * chunk_rows, max_r0)
                pltpu.async_copy(x_hbm.at[pl.ds(nxt, chunk_rows)],
                                 bufs[b2], lsems[b2])
            return 0

        lax.fori_loop(0, n_chunks // _NBUF, group_body, 0)

        for i in range(_AHEAD):
            wait_load((n_chunks + i) % _NBUF)
            wait_store((n_chunks - _AHEAD + i) % _NBUF)

    return scan_k


def kernel(x):
    n_rows, n_cols = x.shape
    scan_k = _make_scan(n_rows, n_cols)
    return scan_k(x)


# speedup vs baseline: 2.8533x; 2.8533x over previous
import functools

import jax
import jax.numpy as jnp
from jax import lax
from jax.experimental import pallas as pl
from jax.experimental.pallas import tpu as pltpu
from jax.experimental.pallas import tpu_sc as plsc

_L = 16
_NBUF = 4
_AHEAD = 2


@functools.lru_cache(maxsize=None)
def _make_scan(n_rows, n_cols, nc=2, ns=16, chunk_rows=4):
    nw = nc * ns
    rows_per_w = n_rows // nw
    n_chunks = rows_per_w // chunk_rows
    assert n_chunks % _NBUF == 0 and n_chunks >= 2 * _NBUF
    vregs_per_row = n_cols // _L
    mesh = plsc.VectorSubcoreMesh(core_axis_name="c", subcore_axis_name="s")

    @functools.partial(
        pl.kernel,
        out_type=jax.ShapeDtypeStruct((n_rows, n_cols), jnp.float32),
        mesh=mesh,
        scratch_types=(
            [pltpu.VMEM((chunk_rows, n_cols), jnp.float32)] * _NBUF
            + [pltpu.SemaphoreType.DMA] * (2 * _NBUF)
        ),
        compiler_params=pltpu.CompilerParams(needs_layout_passes=False),
    )
    def scan_k(x_hbm, out_hbm, *scratch):
        bufs = scratch[:_NBUF]
        lsems = scratch[_NBUF:2 * _NBUF]
        ssems = scratch[2 * _NBUF:]
        wid = lax.axis_index("s") * nc + lax.axis_index("c")
        row_base = wid * rows_per_w
        max_r0 = row_base + (n_chunks - 1) * chunk_rows
        idx15 = jnp.full((_L,), _L - 1, jnp.int32)

        def compute(buf):
            def row_body(r, _):
                def vec_body(j, carry):
                    o = j * _L
                    s = plsc.cumsum(buf[r, pl.ds(o, _L)])
                    buf[r, pl.ds(o, _L)] = s + carry
                    return carry + s.at[idx15].get(mode="promise_in_bounds")

                lax.fori_loop(0, vregs_per_row, vec_body,
                              jnp.zeros((_L,), jnp.float32), unroll=16)
                return 0

            lax.fori_loop(0, chunk_rows, row_body, 0)

        def wait_load(b):
            pltpu.make_async_copy(
                x_hbm.at[pl.ds(row_base, chunk_rows)], bufs[b],
                lsems[b]).wait()

        def wait_store(b):
            pltpu.make_async_copy(
                bufs[b], out_hbm.at[pl.ds(row_base, chunk_rows)],
                ssems[b]).wait()

        for b in range(_AHEAD):
            pltpu.async_copy(
                x_hbm.at[pl.ds(row_base + b * chunk_rows, chunk_rows)],
                bufs[b], lsems[b])

        def group_body(k, _):
            for b in range(_NBUF):
                r0 = row_base + (k * _NBUF + b) * chunk_rows
                wait_load(b)
                compute(bufs[b])
                pltpu.async_copy(bufs[b],
                                 out_hbm.at[pl.ds(r0, chunk_rows)], ssems[b])
                b2 = (b + _AHEAD) % _NBUF
                if b + _AHEAD >= _NBUF:
                    wait_store(b2)
                else:
                    @pl.when(k > 0)
                    def _():
                        wait_store(b2)
                nxt = jnp.minimum(r0 + _AHEAD * chunk_rows, max_r0)
                pltpu.async_copy(x_hbm.at[pl.ds(nxt, chunk_rows)],
                                 bufs[b2], lsems[b2])
            return 0

        lax.fori_loop(0, n_chunks // _NBUF, group_body, 0)

        for i in range(_AHEAD):
            wait_load((n_chunks + i) % _NBUF)
            wait_store((n_chunks - _AHEAD + i) % _NBUF)

    return scan_k


def kernel(x):
    n_rows, n_cols = x.shape
    scan_k = _make_scan(n_rows, n_cols)
    return scan_k(x)


# speedup vs baseline: 4.3630x; 1.5291x over previous
import functools

import jax
import jax.numpy as jnp
from jax import lax
from jax.experimental import pallas as pl
from jax.experimental.pallas import tpu as pltpu
from jax.experimental.pallas import tpu_sc as plsc

_L = 16
_NBUF = 4
_AHEAD = 2
_TC_ROWS = 2560
_BM = 256
_BK = 128


@functools.lru_cache(maxsize=None)
def _make_sc_scan(n_rows, n_cols, row0, nc=2, ns=16, chunk_rows=4):
    nw = nc * ns
    sc_rows = n_rows - row0
    rows_per_w = sc_rows // nw
    n_chunks = rows_per_w // chunk_rows
    assert n_chunks % _NBUF == 0 and n_chunks >= 2 * _AHEAD
    vregs_per_row = n_cols // _L
    mesh = plsc.VectorSubcoreMesh(core_axis_name="c", subcore_axis_name="s")

    @functools.partial(
        pl.kernel,
        out_type=jax.ShapeDtypeStruct((sc_rows, n_cols), jnp.float32),
        mesh=mesh,
        scratch_types=(
            [pltpu.VMEM((chunk_rows, n_cols), jnp.float32)] * _NBUF
            + [pltpu.SemaphoreType.DMA] * (2 * _NBUF)
        ),
        compiler_params=pltpu.CompilerParams(needs_layout_passes=False),
    )
    def scan_k(x_hbm, out_hbm, *scratch):
        bufs = scratch[:_NBUF]
        lsems = scratch[_NBUF:2 * _NBUF]
        ssems = scratch[2 * _NBUF:]
        wid = lax.axis_index("s") * nc + lax.axis_index("c")
        out_base = wid * rows_per_w
        max_o0 = out_base + (n_chunks - 1) * chunk_rows
        idx15 = jnp.full((_L,), _L - 1, jnp.int32)

        def compute(buf):
            def row_body(r, _):
                def vec_body(j, carry):
                    o = j * _L
                    s = plsc.cumsum(buf[r, pl.ds(o, _L)])
                    buf[r, pl.ds(o, _L)] = s + carry
                    return carry + s.at[idx15].get(mode="promise_in_bounds")

                lax.fori_loop(0, vregs_per_row, vec_body,
                              jnp.zeros((_L,), jnp.float32), unroll=16)
                return 0

            lax.fori_loop(0, chunk_rows, row_body, 0)

        def wait_load(b):
            pltpu.make_async_copy(
                x_hbm.at[pl.ds(row0, chunk_rows)], bufs[b], lsems[b]).wait()

        def wait_store(b):
            pltpu.make_async_copy(
                bufs[b], out_hbm.at[pl.ds(out_base, chunk_rows)],
                ssems[b]).wait()

        for b in range(_AHEAD):
            pltpu.async_copy(
                x_hbm.at[pl.ds(row0 + out_base + b * chunk_rows, chunk_rows)],
                bufs[b], lsems[b])

        def group_body(k, _):
            for b in range(_NBUF):
                o0 = out_base + (k * _NBUF + b) * chunk_rows
                wait_load(b)
                compute(bufs[b])
                pltpu.async_copy(bufs[b],
                                 out_hbm.at[pl.ds(o0, chunk_rows)], ssems[b])
                b2 = (b + _AHEAD) % _NBUF
                if b + _AHEAD >= _NBUF:
                    wait_store(b2)
                else:
                    @pl.when(k > 0)
                    def _():
                        wait_store(b2)
                nxt = jnp.minimum(o0 + _AHEAD * chunk_rows, max_o0)
                pltpu.async_copy(x_hbm.at[pl.ds(row0 + nxt, chunk_rows)],
                                 bufs[b2], lsems[b2])
            return 0

        lax.fori_loop(0, n_chunks // _NBUF, group_body, 0)

        for i in range(_AHEAD):
            wait_load((n_chunks + i) % _NBUF)
            wait_store((n_chunks - _AHEAD + i) % _NBUF)

    return scan_k


@functools.lru_cache(maxsize=None)
def _make_tc_scan(tc_rows, n_cols):
    n_chunks = n_cols // _BK

    def body(x_ref, o_ref):
        i0 = lax.broadcasted_iota(jnp.int32, (_BK, _BK), 0)
        i1 = lax.broadcasted_iota(jnp.int32, (_BK, _BK), 1)
        tri = (i0 <= i1).astype(jnp.float32)

        carry = jnp.zeros((_BM, _BK), jnp.float32)
        for c in range(n_chunks):
            xc = x_ref[:, c * _BK:(c + 1) * _BK]
            y = jnp.dot(xc, tri, preferred_element_type=jnp.float32) + carry
            o_ref[:, c * _BK:(c + 1) * _BK] = y
            carry = jnp.broadcast_to(y[:, _BK - 1:_BK], (_BM, _BK))

    return pl.pallas_call(
        body,
        grid=(tc_rows // _BM,),
        in_specs=[pl.BlockSpec((_BM, n_cols), lambda m: (m, 0))],
        out_specs=pl.BlockSpec((_BM, n_cols), lambda m: (m, 0)),
        out_shape=jax.ShapeDtypeStruct((tc_rows, n_cols), jnp.float32),
    )


def kernel(x):
    n_rows, n_cols = x.shape
    return _make_tc_scan(n_rows, n_cols)(x)
